# baseline (speedup 1.0000x reference)
import functools

import jax
import jax.numpy as jnp
from jax import lax
from jax.experimental import pallas as pl
from jax.experimental.pallas import tpu as pltpu
from jax.experimental.pallas import tpu_sc as plsc

_HIDDEN = 64
_SCALE = 8.0


@functools.cache
def _sc_info():
    info = plsc.get_sparse_core_info()
    return info.num_cores, info.num_subcores, info.num_lanes


@functools.cache
def _make_compact(V):
    NC, NS, L = _sc_info()
    NW = NC * NS
    W = 2 * _HIDDEN
    nblk = V // W
    tail = V % W
    per_w = nblk // NW + (-(nblk // NW) % 2) + 2
    span = nblk - per_w
    mesh = plsc.VectorSubcoreMesh(core_axis_name="c", subcore_axis_name="s")

    @functools.partial(
        pl.kernel,
        out_type=jax.ShapeDtypeStruct((V // 2, W), jnp.float32),
        mesh=mesh,
        scratch_types=[
            pltpu.VMEM((_HIDDEN, W), jnp.float32),
            pltpu.VMEM((_HIDDEN, W), jnp.float32),
            pltpu.VMEM((_HIDDEN, W), jnp.float32),
            pltpu.VMEM((_HIDDEN, W), jnp.float32),
            pltpu.SemaphoreType.DMA,
            pltpu.SemaphoreType.DMA,
        ],
        compiler_params=pltpu.CompilerParams(needs_layout_passes=False),
    )
    def compact(tt_hbm, t2_hbm, in0, in1, ob0, ob1, so0, so1):
        in_v = (in0, in1)
        out_v = (ob0, ob1)
        osem = (so0, so1)
        wid = lax.axis_index("s") * NC + lax.axis_index("c")
        start_blk = (wid * span) // (NW - 1)
        iota = jax.lax.iota(jnp.int32, L)

        def transpose_block(src, dst):
            def per_pair(p, carry):
                for s in range(2):
                    col = jnp.zeros((L,), jnp.int32) + (2 * p + s)
                    for j in range(_HIDDEN // L):
                        rows = iota + (j * L)
                        dst[p, pl.ds(s * _HIDDEN + j * L, L)] = (
                            plsc.load_gather(src, [rows, col])
                        )
                return carry

            lax.fori_loop(0, W // 2, per_pair, 0)

        def pair_body(m, carry):
            for b in range(2):
                k = start_blk + 2 * m + b
                c0 = k * W

                @pl.when(m >= 1)
                def _():
                    pltpu.make_async_copy(
                        out_v[b], t2_hbm.at[pl.ds(0, W // 2)], osem[b]
                    ).wait()

                pltpu.sync_copy(tt_hbm.at[:, pl.ds(c0, W)], in_v[b])
                transpose_block(in_v[b], out_v[b])
                pltpu.async_copy(
                    out_v[b], t2_hbm.at[pl.ds(k * (W // 2), W // 2)], osem[b]
                )
            return carry

        lax.fori_loop(0, per_w // 2, pair_body, 0)
        for b in range(2):
            pltpu.make_async_copy(
                out_v[b], t2_hbm.at[pl.ds(0, W // 2)], osem[b]
            ).wait()

        if tail:
            @pl.when(wid == 0)
            def _():
                c0 = nblk * W
                for h in range(_HIDDEN):
                    pltpu.sync_copy(
                        tt_hbm.at[h, pl.ds(c0, tail)],
                        in_v[0].at[h, pl.ds(0, tail)],
                    )
                transpose_block(in_v[0], out_v[0])
                pltpu.sync_copy(
                    out_v[0].at[pl.ds(0, tail // 2)],
                    t2_hbm.at[pl.ds(c0 // 2, tail // 2)],
                )

    return compact


@functools.cache
def _make_lookup(B, V2):
    NC, NS, L = _sc_info()
    NW = NC * NS
    assert B % NW == 0
    b_per_w = B // NW
    C = 400
    n_chunks = b_per_w // C
    assert b_per_w % C == 0
    mesh = plsc.VectorSubcoreMesh(core_axis_name="c", subcore_axis_name="s")

    @functools.partial(
        pl.kernel,
        out_type=jax.ShapeDtypeStruct((B, _HIDDEN), jnp.float32),
        mesh=mesh,
        scratch_types=[
            pltpu.VMEM((C,), jnp.int32),
            pltpu.VMEM((C,), jnp.int32),
            pltpu.VMEM((C, 2 * _HIDDEN), jnp.float32),
            pltpu.VMEM((C, _HIDDEN), jnp.float32),
            pltpu.SemaphoreType.DMA,
        ],
    )
    def lookup(idx_hbm, t2_hbm, out_hbm, idx_v, pidx_v, pairs_v,
               stage_v, sem):
        wid = lax.axis_index("s") * NC + lax.axis_index("c")
        base = wid * b_per_w

        def chunk_body(c, carry):
            start = base + c * C
            pltpu.sync_copy(idx_hbm.at[pl.ds(start, C)], idx_v)

            def halve(g, carry2):
                v = idx_v[pl.ds(g * L, L)]
                pidx_v[pl.ds(g * L, L)] = v >> 1
                return carry2

            lax.fori_loop(0, C // L, halve, 0)
            pltpu.async_copy(t2_hbm.at[pidx_v], pairs_v, sem).wait()

            def select_group(g, carry2):
                idx16 = idx_v[pl.ds(g * L, L)]
                off16 = (idx16 & 1) * _HIDDEN
                for l in range(L):
                    off = off16[l]
                    i = g * L + l
                    for j in range(_HIDDEN // L):
                        stage_v[i, pl.ds(j * L, L)] = (
                            pairs_v[i, pl.ds(off + j * L, L)] * _SCALE
                        )
                return carry2

            lax.fori_loop(0, C // L, select_group, 0)
            pltpu.sync_copy(stage_v, out_hbm.at[pl.ds(start, C)])
            return carry

        lax.fori_loop(0, n_chunks, chunk_body, 0)

    return lookup


def kernel(x, table):
    B = x.shape[0] * x.shape[1]
    V = table.shape[0]
    flat = x.reshape(B).astype(jnp.int32)
    tt = table.T
    t2 = _make_compact(V)(tt)
    out = _make_lookup(B, V // 2)(flat, t2)
    return out.reshape(x.shape[0], x.shape[1], _HIDDEN)


# speedup vs baseline: 2.2495x; 2.2495x over previous
import functools

import jax
import jax.numpy as jnp
from jax import lax
from jax.experimental import pallas as pl
from jax.experimental.pallas import tpu as pltpu
from jax.experimental.pallas import tpu_sc as plsc

_HIDDEN = 64
_SCALE = 8.0


@functools.cache
def _make_lookup(B):
    info = plsc.get_sparse_core_info()
    NC, NS, L = info.num_cores, info.num_subcores, info.num_lanes
    NW = NC * NS
    assert B % NW == 0
    b_per_w = B // NW
    C = 800
    n_chunks = b_per_w // C
    assert b_per_w % C == 0

    mesh = plsc.VectorSubcoreMesh(core_axis_name="c", subcore_axis_name="s")

    @functools.partial(
        pl.kernel,
        out_type=jax.ShapeDtypeStruct((B, _HIDDEN), jnp.float32),
        mesh=mesh,
        scratch_types=[
            pltpu.VMEM((C,), jnp.int32),
            pltpu.VMEM((C,), jnp.int32),
            pltpu.VMEM((C, _HIDDEN), jnp.float32),
            pltpu.VMEM((C, _HIDDEN), jnp.float32),
            pltpu.SemaphoreType.DMA,
            pltpu.SemaphoreType.DMA,
            pltpu.SemaphoreType.DMA,
            pltpu.SemaphoreType.DMA,
        ],
        compiler_params=pltpu.CompilerParams(use_tc_tiling_on_sc=False),
    )
    def lookup(idx_hbm, table_hbm, out_hbm, i0, i1, r0, r1, g0, g1, s0, s1):
        idx_v = (i0, i1)
        rows_v = (r0, r1)
        gsem = (g0, g1)
        ssem = (s0, s1)
        wid = lax.axis_index("s") * NC + lax.axis_index("c")
        base = wid * b_per_w

        def start_gather(c, b):
            start = base + c * C
            pltpu.sync_copy(idx_hbm.at[pl.ds(start, C)], idx_v[b])
            return pltpu.async_copy(table_hbm.at[idx_v[b]], rows_v[b], gsem[b])

        def scale(ref):
            @plsc.parallel_loop(0, C, step=1, unroll=8)
            def _(i):
                for j in range(_HIDDEN // L):
                    ref[i, pl.ds(j * L, L)] = ref[i, pl.ds(j * L, L)] * _SCALE

        gathers = {0: start_gather(0, 0)}
        scatters = {}
        for c in range(n_chunks):
            b = c % 2
            gathers[c].wait()
            if c + 1 < n_chunks:
                if c - 1 >= 0:
                    scatters[c - 1].wait()
                gathers[c + 1] = start_gather(c + 1, 1 - b)
            scale(rows_v[b])
            scatters[c] = pltpu.async_copy(
                rows_v[b], out_hbm.at[pl.ds(base + c * C, C)], ssem[b]
            )
        if n_chunks >= 2:
            scatters[n_chunks - 2].wait()
        scatters[n_chunks - 1].wait()

    return lookup


def kernel(x, table):
    B = x.shape[0] * x.shape[1]
    flat = x.reshape(B).astype(jnp.int32)
    out = _make_lookup(B)(flat, table)
    return out.reshape(x.shape[0], x.shape[1], _HIDDEN)
